# Initial kernel scaffold; baseline (speedup 1.0000x reference)
import functools

import jax
import jax.numpy as jnp
from jax import lax
from jax.experimental import pallas as pl
from jax.experimental.pallas import tpu as pltpu
from jax.experimental.pallas import tpu_sc as plsc

N = 10000
E = 320000
NC = 2
NS = 16
NW = NC * NS
EW = E // NW
CHUNK = 2000
NCHUNK = EW // CHUNK
IROW = 125
ROWS = CHUNK // IROW
GROUPS = CHUNK // 16
ER = E // IROW


def _edge_body(H, C, src_hbm, dst_hbm, xl_hbm, xr_hbm, att_hbm, zero_hbm,
               out_hbm, sidx, didx, xl_buf, xr_buf, sc_buf, att_sm, acc):
    c = lax.axis_index("c")
    s = lax.axis_index("s")
    wid = s * NC + c

    @pl.when(s == 0)
    def _():
        pltpu.sync_copy(zero_hbm, acc)

    pltpu.sync_copy(att_hbm, att_sm)
    plsc.subcore_barrier()

    att_vec = [jnp.broadcast_to(att_sm[ch], (16,)) for ch in range(16)]
    iota16 = lax.iota(jnp.int32, 16)

    for k in range(NCHUNK):
        rowbase = wid * (EW // IROW) + k * ROWS
        pltpu.sync_copy(src_hbm.at[pl.ds(rowbase, ROWS)], sidx)
        pltpu.sync_copy(dst_hbm.at[pl.ds(rowbase, ROWS)], didx)
        for j in range(ROWS):
            pltpu.sync_copy(xl_hbm.at[sidx.at[j]],
                            xl_buf.at[pl.ds(j * IROW, IROW)])
            pltpu.sync_copy(xr_hbm.at[didx.at[j]],
                            xr_buf.at[pl.ds(j * IROW, IROW)])

        def group(g, carry):
            row = g * 16 + iota16
            accs = [jnp.zeros((16,), jnp.float32) for _ in range(H)]
            vals = []
            for ch in range(16):
                colv = jnp.full((16,), ch, jnp.int32)
                l = plsc.load_gather(xl_buf, [row, colv])
                r = plsc.load_gather(xr_buf, [row, colv])
                t = l + r
                e = jnp.maximum(t, 0.2 * t)
                accs[ch // C] = accs[ch // C] + att_vec[ch] * e
                vals.append(l)
            ws = [jnp.exp(a) for a in accs]
            for ch in range(16):
                colv = jnp.full((16,), ch, jnp.int32)
                plsc.store_scatter(sc_buf, [row, colv], ws[ch // C] * vals[ch])
            for h in range(H):
                colv = jnp.full((16,), 16 + h, jnp.int32)
                plsc.store_scatter(sc_buf, [row, colv], ws[h])
            return carry

        lax.fori_loop(0, GROUPS, group, 0)

        for j in range(ROWS):
            pltpu.sync_copy(sc_buf.at[pl.ds(j * IROW, IROW)],
                            acc.at[didx.at[j]], add=True)

    plsc.subcore_barrier()

    @pl.when(s == 0)
    def _():
        pltpu.sync_copy(acc, out_hbm.at[c])


def _make_edge_pass(H):
    C = 16 // H
    R = 16 + H
    return pl.kernel(
        functools.partial(_edge_body, H, C),
        out_type=jax.ShapeDtypeStruct((NC, N, R), jnp.float32),
        mesh=plsc.VectorSubcoreMesh(core_axis_name="c", subcore_axis_name="s"),
        scratch_types=[
            pltpu.VMEM((ROWS, IROW), jnp.int32),
            pltpu.VMEM((ROWS, IROW), jnp.int32),
            pltpu.VMEM((CHUNK, 16), jnp.float32),
            pltpu.VMEM((CHUNK, 16), jnp.float32),
            pltpu.VMEM((CHUNK, R), jnp.float32),
            pltpu.SMEM((16,), jnp.float32),
            pltpu.VMEM_SHARED((N, R), jnp.float32),
        ],
        name=f"gat_edge_pass_h{H}",
    )


_edge_pass_l1 = _make_edge_pass(2)
_edge_pass_l2 = _make_edge_pass(1)


def _dense_body(x_ref, wl_ref, bl_ref, wr_ref, br_ref, xl_ref, xr_ref):
    x = x_ref[...]
    xl_ref[...] = jnp.dot(x, wl_ref[...],
                          preferred_element_type=jnp.float32) + bl_ref[...]
    xr_ref[...] = jnp.dot(x, wr_ref[...],
                          preferred_element_type=jnp.float32) + br_ref[...]


_dense1 = pl.pallas_call(
    _dense_body,
    out_shape=(jax.ShapeDtypeStruct((N, 16), jnp.float32),
               jax.ShapeDtypeStruct((N, 16), jnp.float32)),
)


def _fin1_body(p_ref, b1_ref, wl_ref, bl_ref, wr_ref, br_ref, xl_ref, xr_ref):
    num = p_ref[0, :, :16] + p_ref[1, :, :16]
    d0 = p_ref[0, :, 16:17] + p_ref[1, :, 16:17]
    d1 = p_ref[0, :, 17:18] + p_ref[1, :, 17:18]
    den = jnp.concatenate([jnp.broadcast_to(d0, (N, 8)),
                           jnp.broadcast_to(d1, (N, 8))], axis=1)
    h = num / (den + 1e-16) + b1_ref[...]
    h = jnp.where(h > 0, h, jnp.expm1(h))
    xl_ref[...] = jnp.dot(h, wl_ref[...],
                          preferred_element_type=jnp.float32) + bl_ref[...]
    xr_ref[...] = jnp.dot(h, wr_ref[...],
                          preferred_element_type=jnp.float32) + br_ref[...]


_fin1 = pl.pallas_call(
    _fin1_body,
    out_shape=(jax.ShapeDtypeStruct((N, 16), jnp.float32),
               jax.ShapeDtypeStruct((N, 16), jnp.float32)),
)


def _fin2_body(p_ref, b2_ref, wlin_ref, blin_ref, out_ref):
    num = p_ref[0, :, :16] + p_ref[1, :, :16]
    den = p_ref[0, :, 16:17] + p_ref[1, :, 16:17]
    h = num / (den + 1e-16) + b2_ref[...]
    out_ref[...] = jnp.dot(h, wlin_ref[...],
                           preferred_element_type=jnp.float32) + blin_ref[...]


_fin2 = pl.pallas_call(
    _fin2_body,
    out_shape=jax.ShapeDtypeStruct((N, 1), jnp.float32),
)


@jax.jit
def kernel(x, edge_index, W1l, b1l, W1r, b1r, att1, bias1,
           W2l, b2l, W2r, b2r, att2, bias2, Wlin, blin):
    src = edge_index[0].reshape(ER, IROW)
    dst = edge_index[1].reshape(ER, IROW)
    z18 = jnp.zeros((N, 18), jnp.float32)
    z17 = jnp.zeros((N, 17), jnp.float32)

    xl1, xr1 = _dense1(x, W1l, b1l, W1r, b1r)
    p1 = _edge_pass_l1(src, dst, xl1, xr1, att1.reshape(16), z18)
    xl2, xr2 = _fin1(p1, bias1, W2l, b2l, W2r, b2r)
    p2 = _edge_pass_l2(src, dst, xl2, xr2, att2.reshape(16), z17)
    return _fin2(p2, bias2, Wlin, blin)


# speedup vs baseline: 56.1822x; 56.1822x over previous
import functools

import jax
import jax.numpy as jnp
from jax import lax
from jax.experimental import pallas as pl
from jax.experimental.pallas import tpu as pltpu
from jax.experimental.pallas import tpu_sc as plsc

N = 10000
NP = N + 16
E = 320000
NC = 2
NS = 16
NW = NC * NS
IROW = 128
EP = 327680
EW = EP // NW
CHUNK = 1024
NCHUNK = EW // CHUNK
ROWS = CHUNK // IROW
GROUPS = CHUNK // 16
ER = EP // IROW
RW = EW // IROW
R = 32


def _edge_body(H, C, src_hbm, dst_hbm, xl_hbm, xr_hbm, att_hbm, zero_hbm,
               out_hbm, sidx, didx, xl_buf, xr_buf, sc_buf, att_vm, acc):
    c = lax.axis_index("c")
    s = lax.axis_index("s")
    wid = s * NC + c

    @pl.when(s == 0)
    def _():
        pltpu.sync_copy(zero_hbm, acc)

    pltpu.sync_copy(zero_hbm.at[pl.ds(0, CHUNK)], sc_buf)
    pltpu.sync_copy(att_hbm, att_vm)
    plsc.subcore_barrier()

    att_all = att_vm[...]
    att_vec = [jnp.broadcast_to(att_all[ch], (16,)) for ch in range(16)]
    iota16 = lax.iota(jnp.int32, 16)

    for k in range(NCHUNK):
        rowbase = wid * RW + k * ROWS
        pltpu.sync_copy(src_hbm.at[pl.ds(rowbase, ROWS)], sidx)
        pltpu.sync_copy(dst_hbm.at[pl.ds(rowbase, ROWS)], didx)
        for j in range(ROWS):
            pltpu.sync_copy(xl_hbm.at[sidx.at[j]],
                            xl_buf.at[pl.ds(j * IROW, IROW)])
            pltpu.sync_copy(xr_hbm.at[didx.at[j]],
                            xr_buf.at[pl.ds(j * IROW, IROW)])

        def group(g, carry):
            row = g * 16 + iota16
            accs = [jnp.zeros((16,), jnp.float32) for _ in range(H)]
            vals = []
            for ch in range(16):
                colv = jnp.full((16,), ch, jnp.int32)
                l = plsc.load_gather(xl_buf, [row, colv])
                r = plsc.load_gather(xr_buf, [row, colv])
                t = l + r
                e = jnp.maximum(t, 0.2 * t)
                accs[ch // C] = accs[ch // C] + att_vec[ch] * e
                vals.append(l)
            ws = [jnp.exp(a) for a in accs]
            for ch in range(16):
                colv = jnp.full((16,), ch, jnp.int32)
                plsc.store_scatter(sc_buf, [row, colv], ws[ch // C] * vals[ch])
            for h in range(H):
                colv = jnp.full((16,), 16 + h, jnp.int32)
                plsc.store_scatter(sc_buf, [row, colv], ws[h])
            return carry

        lax.fori_loop(0, GROUPS, group, 0)

        for j in range(ROWS):
            pltpu.sync_copy(sc_buf.at[pl.ds(j * IROW, IROW)],
                            acc.at[didx.at[j]], add=True)

    plsc.subcore_barrier()

    @pl.when(s == 0)
    def _():
        pltpu.sync_copy(acc, out_hbm.at[c])


def _make_edge_pass(H):
    C = 16 // H
    return pl.kernel(
        functools.partial(_edge_body, H, C),
        out_type=jax.ShapeDtypeStruct((NC, NP, R), jnp.float32),
        mesh=plsc.VectorSubcoreMesh(core_axis_name="c", subcore_axis_name="s"),
        compiler_params=pltpu.CompilerParams(needs_layout_passes=False,
                                            use_tc_tiling_on_sc=False),
        scratch_types=[
            pltpu.VMEM((ROWS, IROW), jnp.int32),
            pltpu.VMEM((ROWS, IROW), jnp.int32),
            pltpu.VMEM((CHUNK, 16), jnp.float32),
            pltpu.VMEM((CHUNK, 16), jnp.float32),
            pltpu.VMEM((CHUNK, R), jnp.float32),
            pltpu.VMEM((16,), jnp.float32),
            pltpu.VMEM_SHARED((NP, R), jnp.float32),
        ],
        name=f"gat_edge_pass_h{H}",
    )


_edge_pass_l1 = _make_edge_pass(2)
_edge_pass_l2 = _make_edge_pass(1)


def _dense_body(x_ref, wl_ref, bl_ref, wr_ref, br_ref, xl_ref, xr_ref):
    x = x_ref[...]
    xl_ref[...] = jnp.dot(x, wl_ref[...],
                          preferred_element_type=jnp.float32) + bl_ref[...]
    xr_ref[...] = jnp.dot(x, wr_ref[...],
                          preferred_element_type=jnp.float32) + br_ref[...]


_dense1 = pl.pallas_call(
    _dense_body,
    out_shape=(jax.ShapeDtypeStruct((N, 16), jnp.float32),
               jax.ShapeDtypeStruct((N, 16), jnp.float32)),
)


def _fin1_body(p_ref, b1_ref, wl_ref, bl_ref, wr_ref, br_ref, xl_ref, xr_ref):
    num = p_ref[0, :N, :16] + p_ref[1, :N, :16]
    d0 = p_ref[0, :N, 16:17] + p_ref[1, :N, 16:17]
    d1 = p_ref[0, :N, 17:18] + p_ref[1, :N, 17:18]
    den = jnp.concatenate([jnp.broadcast_to(d0, (N, 8)),
                           jnp.broadcast_to(d1, (N, 8))], axis=1)
    h = num / (den + 1e-16) + b1_ref[...]
    h = jnp.where(h > 0, h, jnp.exp(jnp.minimum(h, 0.0)) - 1.0)
    xl_ref[...] = jnp.dot(h, wl_ref[...],
                          preferred_element_type=jnp.float32) + bl_ref[...]
    xr_ref[...] = jnp.dot(h, wr_ref[...],
                          preferred_element_type=jnp.float32) + br_ref[...]


_fin1 = pl.pallas_call(
    _fin1_body,
    out_shape=(jax.ShapeDtypeStruct((N, 16), jnp.float32),
               jax.ShapeDtypeStruct((N, 16), jnp.float32)),
)


def _fin2_body(p_ref, b2_ref, wlin_ref, blin_ref, out_ref):
    num = p_ref[0, :N, :16] + p_ref[1, :N, :16]
    den = p_ref[0, :N, 16:17] + p_ref[1, :N, 16:17]
    h = num / (den + 1e-16) + b2_ref[...]
    out_ref[...] = jnp.dot(h, wlin_ref[...],
                           preferred_element_type=jnp.float32) + blin_ref[...]


_fin2 = pl.pallas_call(
    _fin2_body,
    out_shape=jax.ShapeDtypeStruct((N, 1), jnp.float32),
)


@jax.jit
def kernel(x, edge_index, W1l, b1l, W1r, b1r, att1, bias1,
           W2l, b2l, W2r, b2r, att2, bias2, Wlin, blin):
    src = jnp.concatenate(
        [edge_index[0], jnp.zeros((EP - E,), jnp.int32)]).reshape(ER, IROW)
    dst = jnp.concatenate(
        [edge_index[1], jnp.full((EP - E,), N, jnp.int32)]).reshape(ER, IROW)
    zeros = jnp.zeros((NP, R), jnp.float32)
    pad16 = jnp.zeros((16, 16), jnp.float32)

    xl1, xr1 = _dense1(x, W1l, b1l, W1r, b1r)
    xr1p = jnp.concatenate([xr1, pad16])
    p1 = _edge_pass_l1(src, dst, xl1, xr1p, att1.reshape(16), zeros)
    xl2, xr2 = _fin1(p1, bias1, W2l, b2l, W2r, b2r)
    xr2p = jnp.concatenate([xr2, pad16])
    p2 = _edge_pass_l2(src, dst, xl2, xr2p, att2.reshape(16), zeros)
    return _fin2(p2, bias2, Wlin, blin)
